# baseline (speedup 1.0000x reference)
import functools

import jax
import jax.numpy as jnp
from jax import lax
from jax.experimental import pallas as pl
from jax.experimental.pallas import tpu as pltpu
from jax.experimental.pallas import tpu_sc as plsc

B = 16384
D = 32
NC = 2
NS = 16
NW = NC * NS
BPW = B // NW
CHUNK = 128
NCHUNK = BPW // CHUNK
GROUPS = BPW // 16


def kernel(user_id, item_id, user_table, item_table):
    mesh = plsc.VectorSubcoreMesh(core_axis_name="c", subcore_axis_name="s")

    @functools.partial(
        pl.kernel,
        mesh=mesh,
        out_type=jax.ShapeDtypeStruct((B,), jnp.float32),
        compiler_params=pltpu.CompilerParams(
            needs_layout_passes=False, use_tc_tiling_on_sc=False),
        scratch_types=[
            pltpu.VMEM((NCHUNK, CHUNK), jnp.int32),
            pltpu.VMEM((NCHUNK, CHUNK), jnp.int32),
            pltpu.VMEM((BPW, D), jnp.float32),
            pltpu.VMEM((BPW, D), jnp.float32),
            pltpu.VMEM((BPW * 16,), jnp.float32),
            pltpu.VMEM((BPW,), jnp.float32),
            pltpu.SemaphoreType.DMA,
        ],
    )
    def run(uid_hbm, iid_hbm, ut_hbm, it_hbm, out_hbm,
            uidx_v, iidx_v, urows_v, irows_v, part_v, out_v, sem):
        wid = lax.axis_index("s") * NC + lax.axis_index("c")
        base = wid * BPW

        for j in range(NCHUNK):
            pltpu.sync_copy(uid_hbm.at[pl.ds(base + j * CHUNK, CHUNK)],
                            uidx_v.at[j])
            pltpu.sync_copy(iid_hbm.at[pl.ds(base + j * CHUNK, CHUNK)],
                            iidx_v.at[j])

        copies = []
        for j in range(NCHUNK):
            copies.append(pltpu.async_copy(
                ut_hbm.at[uidx_v.at[j]],
                urows_v.at[pl.ds(j * CHUNK, CHUNK)], sem))
            copies.append(pltpu.async_copy(
                it_hbm.at[iidx_v.at[j]],
                irows_v.at[pl.ds(j * CHUNK, CHUNK)], sem))
        for c in copies:
            c.wait()

        lane = lax.iota(jnp.int32, 16)

        def row_body(r, carry):
            u0 = urows_v[r, pl.ds(0, 16)]
            u1 = urows_v[r, pl.ds(16, 16)]
            i0 = irows_v[r, pl.ds(0, 16)]
            i1 = irows_v[r, pl.ds(16, 16)]
            part_v[pl.ds(r * 16, 16)] = u0 * i0 + u1 * i1
            return carry

        lax.fori_loop(0, BPW, row_body, 0)

        def group_body(g, carry):
            row_idx = g * 16 + lane
            flat_base = row_idx * 16
            acc = jnp.zeros((16,), jnp.float32)
            for c in range(16):
                acc = acc + plsc.load_gather(part_v, [flat_base + c])
            plsc.store_scatter(out_v, [row_idx], acc)
            return carry

        lax.fori_loop(0, GROUPS, group_body, 0)

        pltpu.sync_copy(out_v, out_hbm.at[pl.ds(base, BPW)])

    return run(user_id, item_id, user_table, item_table)
